# baseline (speedup 1.0000x reference)
import functools
import math

import jax
import jax.numpy as jnp
from jax import lax
from jax.experimental import pallas as pl
from jax.experimental.pallas import tpu as pltpu
from jax.experimental.pallas import tpu_sc as plsc

_K = 10
_GROUPS = 4
_EPS = 1e-5
_GELU_C = math.sqrt(2.0 / math.pi)
_SC_ROWS = 1152
_NWORKERS = 32


def _gelu(x):
    return 0.5 * x * (1.0 + jnp.tanh(_GELU_C * (x + 0.044715 * x ** 3)))


def _dot(a, b, precision=lax.Precision.DEFAULT):
    return lax.dot_general(a, b, (((1,), (0,)), ((), ())),
                           preferred_element_type=jnp.float32,
                           precision=precision)


def _dotb(a, b):
    return lax.dot_general(a.astype(jnp.bfloat16), b.astype(jnp.bfloat16),
                           (((1,), (0,)), ((), ())),
                           preferred_element_type=jnp.float32)


def _gn_gelu(h, gamma, beta):
    C, N = h.shape
    gs = C // _GROUPS
    outs = []
    for g in range(_GROUPS):
        seg = lax.slice(h, (g * gs, 0), ((g + 1) * gs, N))
        mu = jnp.mean(seg)
        cen = seg - mu
        var = jnp.mean(cen * cen)
        outs.append(cen * lax.rsqrt(var + _EPS))
    hn = jnp.concatenate(outs, axis=0)
    return _gelu(hn * gamma + beta)


def _nbr_sum_tc(xt3, coords, n, row_lo, r):
    xx = jnp.sum(xt3 * xt3, axis=1, keepdims=True)
    accs = []
    for t in range((n - row_lo) // r):
        lo = row_lo + t * r
        cr = lax.slice(coords, (0, lo), (3, lo + r))
        xxr = jnp.sum(cr * cr, axis=0, keepdims=True)
        inner = jnp.float32(-2.0) * _dot(xt3, cr)
        key = (-xxr) - inner - xx

        def body(i, carry):
            key, acc = carry
            m = jnp.max(key, axis=0, keepdims=True)
            eq = key == m
            oh = jnp.where(eq, jnp.float32(1.0), jnp.float32(0.0))
            acc = acc + _dot(coords, oh)
            key = jnp.where(eq, jnp.float32(-3e38), key)
            return key, acc

        _, acc = lax.fori_loop(0, _K, body,
                               (key, jnp.zeros((3, r), jnp.float32)))
        accs.append(acc)
    return jnp.concatenate(accs, axis=1)


def _tc_knn_kernel(xt_ref, x_ref, out_ref, *, n, row_lo, r):
    xt = xt_ref[0]
    x6 = x_ref[0]
    for h in range(2):
        xt3 = lax.slice(xt, (0, 3 * h), (n, 3 * h + 3))
        c3 = lax.slice(x6, (3 * h, 0), (3 * h + 3, n))
        out_ref[0, h] = _nbr_sum_tc(xt3, c3, n, row_lo, r)


def _sc_knn(xs, n):
    ns = _SC_ROWS
    nrw = ns * 16 // _NWORKERS
    nchunks = n // 16
    mesh = plsc.VectorSubcoreMesh(core_axis_name="c", subcore_axis_name="s")

    @functools.partial(
        pl.kernel, mesh=mesh,
        compiler_params=pltpu.CompilerParams(needs_layout_passes=False),
        out_type=jax.ShapeDtypeStruct((16 * 3 * ns,), jnp.float32),
        scratch_types=[
            pltpu.VMEM((n,), jnp.float32),
            pltpu.VMEM((n,), jnp.float32),
            pltpu.VMEM((n,), jnp.float32),
            pltpu.VMEM((n,), jnp.float32),
            pltpu.VMEM((n,), jnp.float32),
            pltpu.VMEM((n,), jnp.float32),
            pltpu.VMEM((n,), jnp.float32),
            pltpu.VMEM((nrw,), jnp.float32),
            pltpu.VMEM((nrw,), jnp.float32),
            pltpu.VMEM((nrw,), jnp.float32),
        ],
    )
    def k(xs_hbm, out_hbm, ce0, ce1, ce2, cb0, cb1, cb2, xxv, o0, o1, o2):
        wid = lax.axis_index("s") * 2 + lax.axis_index("c")
        bh = wid // 2
        sub = wid % 2
        row0 = sub * nrw
        pltpu.sync_copy(xs_hbm.at[pl.ds((bh * 3 + 0) * n, n)], ce0)
        pltpu.sync_copy(xs_hbm.at[pl.ds((bh * 3 + 1) * n, n)], ce1)
        pltpu.sync_copy(xs_hbm.at[pl.ds((bh * 3 + 2) * n, n)], ce2)

        def rnbf16(v):
            bits = plsc.bitcast(v, jnp.uint32)
            r = bits + jnp.uint32(0x7FFF) + ((bits >> 16) & jnp.uint32(1))
            return plsc.bitcast(r & jnp.uint32(0xFFFF0000), jnp.float32)

        def pre(i, _):
            ds = pl.ds(i * 16, 16)
            v0 = ce0[ds]
            v1 = ce1[ds]
            v2 = ce2[ds]
            cb0[ds] = rnbf16(v0)
            cb1[ds] = rnbf16(v1)
            cb2[ds] = rnbf16(v2)
            xxv[ds] = v0 * v0 + v1 * v1 + v2 * v2
            return 0

        lax.fori_loop(0, nchunks, pre, 0)

        lanes = lax.broadcasted_iota(jnp.int32, (16,), 0)
        kmask = lanes < _K
        zero = jnp.float32(0.0)

        neg = jnp.full((16,), -3e38, jnp.float32)
        zi = jnp.zeros((16,), jnp.int32)

        def group_body(g, _):
            def row_body(j, res):
                res0, res1, res2 = res
                base = row0 + g * 16 + 2 * j
                rra = jnp.full((16,), base, jnp.int32)
                rrb = jnp.full((16,), base + 1, jnp.int32)
                vxa0 = plsc.load_gather(cb0, [rra])
                vxa1 = plsc.load_gather(cb1, [rra])
                vxa2 = plsc.load_gather(cb2, [rra])
                nxxa = -plsc.load_gather(xxv, [rra])
                vxb0 = plsc.load_gather(cb0, [rrb])
                vxb1 = plsc.load_gather(cb1, [rrb])
                vxb2 = plsc.load_gather(cb2, [rrb])
                nxxb = -plsc.load_gather(xxv, [rrb])

                def merge16(vals, idxs, key, cols):
                    sk, si = plsc.sort_key_val(key, cols, descending=True)
                    rk = lax.rev(sk, (0,))
                    ri = lax.rev(si, (0,))
                    take = vals >= rk
                    nv = jnp.where(take, vals, rk)
                    ni = jnp.where(take, idxs, ri)
                    nv, ni = plsc.sort_key_val(nv, ni, descending=True)
                    return nv, ni

                def chunk_body(cc, carry):
                    va, ia, vb, ib = carry
                    ds = pl.ds(cc * 16, 16)
                    c0 = cb0[ds]
                    c1 = cb1[ds]
                    c2 = cb2[ds]
                    xc = xxv[ds]
                    cols = lanes + cc * 16
                    keya = nxxa - jnp.float32(-2.0) * (
                        vxa0 * c0 + vxa1 * c1 + vxa2 * c2) - xc
                    keyb = nxxb - jnp.float32(-2.0) * (
                        vxb0 * c0 + vxb1 * c1 + vxb2 * c2) - xc
                    va, ia = merge16(va, ia, keya, cols)
                    vb, ib = merge16(vb, ib, keyb, cols)
                    return va, ia, vb, ib

                va, ia, vb, ib = lax.fori_loop(
                    0, nchunks, chunk_body, (neg, zi, neg, zi))

                for (idxs, isj) in ((ia, lanes == 2 * j),
                                    (ib, lanes == 2 * j + 1)):
                    g0 = plsc.load_gather(ce0, [idxs])
                    g1 = plsc.load_gather(ce1, [idxs])
                    g2 = plsc.load_gather(ce2, [idxs])
                    res0 = jnp.where(
                        isj, jnp.sum(jnp.where(kmask, g0, zero)), res0)
                    res1 = jnp.where(
                        isj, jnp.sum(jnp.where(kmask, g1, zero)), res1)
                    res2 = jnp.where(
                        isj, jnp.sum(jnp.where(kmask, g2, zero)), res2)
                return res0, res1, res2

            res0, res1, res2 = lax.fori_loop(
                0, 8, row_body,
                (jnp.zeros((16,), jnp.float32),
                 jnp.zeros((16,), jnp.float32),
                 jnp.zeros((16,), jnp.float32)))
            ds = pl.ds(g * 16, 16)
            o0[ds] = res0
            o1[ds] = res1
            o2[ds] = res2
            return 0

        lax.fori_loop(0, nrw // 16, group_body, 0)

        pltpu.sync_copy(o0, out_hbm.at[pl.ds((bh * 3 + 0) * ns + row0, nrw)])
        pltpu.sync_copy(o1, out_hbm.at[pl.ds((bh * 3 + 1) * ns + row0, nrw)])
        pltpu.sync_copy(o2, out_hbm.at[pl.ds((bh * 3 + 2) * ns + row0, nrw)])

    return k(xs)


def _edge_head(nm, c3, w_c, b_c):
    wa = lax.slice(w_c, (0, 0), (w_c.shape[0], 3))
    wb = lax.slice(w_c, (0, 3), (w_c.shape[0], 6))
    return _dot(wa, nm) + _dot(wb - wa, c3) + b_c


def _mlp3(h, w1, b1, g1, be1, w2, b2, g2, be2, w3, b3):
    h = _gn_gelu(_dotb(w1, h) + b1, g1, be1)
    h = _gn_gelu(_dotb(w2, h) + b2, g2, be2)
    return _dotb(w3, h) + b3


def _dense_kernel(x_ref, scs_ref, tcs_ref, wcd_ref, bcd_ref, wcm_ref, bcm_ref,
                  wd1_ref, bd1_ref, gd1_ref, bed1_ref,
                  wd2_ref, bd2_ref, gd2_ref, bed2_ref, wd3_ref, bd3_ref,
                  wm1_ref, bm1_ref, gm1_ref, bem1_ref,
                  wm2_ref, bm2_ref, gm2_ref, bem2_ref, wm3_ref, bm3_ref,
                  wg1_ref, bg1_ref, gg1_ref, beg1_ref, wg2_ref, bg2_ref,
                  out_ref, *, n):
    x6 = x_ref[0]
    inv_k = jnp.float32(1.0 / _K)
    nms = []
    for h in range(2):
        nm = jnp.concatenate([scs_ref[0, h], tcs_ref[0, h]], axis=1) * inv_k
        nms.append(nm)

    cd = lax.slice(x6, (0, 0), (3, n))
    cm = lax.slice(x6, (3, 0), (6, n))
    xd = _edge_head(nms[0], cd, wcd_ref[...], bcd_ref[...])
    xm = _edge_head(nms[1], cm, wcm_ref[...], bcm_ref[...])

    xdo = _mlp3(xd, wd1_ref[...], bd1_ref[...], gd1_ref[...], bed1_ref[...],
                wd2_ref[...], bd2_ref[...], gd2_ref[...], bed2_ref[...],
                wd3_ref[...], bd3_ref[...])
    xmo = _mlp3(xm, wm1_ref[...], bm1_ref[...], gm1_ref[...], bem1_ref[...],
                wm2_ref[...], bm2_ref[...], gm2_ref[...], bem2_ref[...],
                wm3_ref[...], bm3_ref[...])

    xc = jnp.concatenate([xdo, xmo], axis=0)
    h = _gn_gelu(_dotb(wg1_ref[...], xc) + bg1_ref[...],
                 gg1_ref[...], beg1_ref[...])
    out_ref[0] = _dotb(wg2_ref[...], h) + bg2_ref[...]


def kernel(x, w_cd, b_cd, w_cm, b_cm, w_d1, b_d1, g_d1, be_d1, w_d2, b_d2,
           g_d2, be_d2, w_d3, b_d3, w_m1, b_m1, g_m1, be_m1, w_m2, b_m2,
           g_m2, be_m2, w_m3, b_m3, w_g1, b_g1, g_g1, be_g1, w_g2, b_g2):
    B, C, N = x.shape
    ns = _SC_ROWS
    nt = N - ns
    R = nt // -(-nt // 1024)
    xt = jnp.transpose(x, (0, 2, 1))
    xs = x.reshape(B * 2 * 3 * N)

    scsum = _sc_knn(xs, N).reshape(B, 2, 3, ns)

    tcsum = pl.pallas_call(
        functools.partial(_tc_knn_kernel, n=N, row_lo=ns, r=R),
        grid=(B,),
        in_specs=[
            pl.BlockSpec((1, N, C), lambda b: (b, 0, 0)),
            pl.BlockSpec((1, C, N), lambda b: (b, 0, 0)),
        ],
        out_specs=pl.BlockSpec((1, 2, 3, nt), lambda b: (b, 0, 0, 0)),
        out_shape=jax.ShapeDtypeStruct((B, 2, 3, nt), jnp.float32),
        compiler_params=pltpu.CompilerParams(
            dimension_semantics=("arbitrary",)),
    )(xt, x)

    def col(v):
        return v.reshape(-1, 1)

    weights = (w_cd, col(b_cd), w_cm, col(b_cm),
               w_d1, col(b_d1), col(g_d1), col(be_d1),
               w_d2, col(b_d2), col(g_d2), col(be_d2), w_d3, col(b_d3),
               w_m1, col(b_m1), col(g_m1), col(be_m1),
               w_m2, col(b_m2), col(g_m2), col(be_m2), w_m3, col(b_m3),
               w_g1, col(b_g1), col(g_g1), col(be_g1), w_g2, col(b_g2))

    def wspec(v):
        nd = v.ndim
        return pl.BlockSpec(v.shape, lambda b, _nd=nd: (0,) * _nd)

    in_specs = [
        pl.BlockSpec((1, C, N), lambda b: (b, 0, 0)),
        pl.BlockSpec((1, 2, 3, ns), lambda b: (b, 0, 0, 0)),
        pl.BlockSpec((1, 2, 3, nt), lambda b: (b, 0, 0, 0)),
    ] + [wspec(v) for v in weights]

    out = pl.pallas_call(
        functools.partial(_dense_kernel, n=N),
        grid=(B,),
        in_specs=in_specs,
        out_specs=pl.BlockSpec((1, 512, N), lambda b: (b, 0, 0)),
        out_shape=jax.ShapeDtypeStruct((B, 512, N), jnp.float32),
        compiler_params=pltpu.CompilerParams(
            dimension_semantics=("arbitrary",)),
    )(x, scsum, tcsum, *weights)
    return out


# speedup vs baseline: 1.1225x; 1.1225x over previous
import functools
import math

import jax
import jax.numpy as jnp
from jax import lax
from jax.experimental import pallas as pl
from jax.experimental.pallas import tpu as pltpu
from jax.experimental.pallas import tpu_sc as plsc

_K = 10
_GROUPS = 4
_EPS = 1e-5
_GELU_C = math.sqrt(2.0 / math.pi)
_SC_ROWS = 1280
_NWORKERS = 32


def _gelu(x):
    return 0.5 * x * (1.0 + jnp.tanh(_GELU_C * (x + 0.044715 * x ** 3)))


def _dot(a, b, precision=lax.Precision.DEFAULT):
    return lax.dot_general(a, b, (((1,), (0,)), ((), ())),
                           preferred_element_type=jnp.float32,
                           precision=precision)


def _dotb(a, b):
    return lax.dot_general(a.astype(jnp.bfloat16), b.astype(jnp.bfloat16),
                           (((1,), (0,)), ((), ())),
                           preferred_element_type=jnp.float32)


def _gn_gelu(h, gamma, beta):
    C, N = h.shape
    gs = C // _GROUPS
    outs = []
    for g in range(_GROUPS):
        seg = lax.slice(h, (g * gs, 0), ((g + 1) * gs, N))
        mu = jnp.mean(seg)
        cen = seg - mu
        var = jnp.mean(cen * cen)
        outs.append(cen * lax.rsqrt(var + _EPS))
    hn = jnp.concatenate(outs, axis=0)
    return _gelu(hn * gamma + beta)


def _nbr_sum_tc(xt3, coords, n, row_lo, r):
    xx = jnp.sum(xt3 * xt3, axis=1, keepdims=True)
    accs = []
    for t in range((n - row_lo) // r):
        lo = row_lo + t * r
        cr = lax.slice(coords, (0, lo), (3, lo + r))
        xxr = jnp.sum(cr * cr, axis=0, keepdims=True)
        inner = jnp.float32(-2.0) * _dot(xt3, cr)
        key = (-xxr) - inner - xx

        def body(i, carry):
            key, acc = carry
            m = jnp.max(key, axis=0, keepdims=True)
            eq = key == m
            oh = jnp.where(eq, jnp.float32(1.0), jnp.float32(0.0))
            acc = acc + _dot(coords, oh)
            key = jnp.where(eq, jnp.float32(-3e38), key)
            return key, acc

        _, acc = lax.fori_loop(0, _K, body,
                               (key, jnp.zeros((3, r), jnp.float32)))
        accs.append(acc)
    return jnp.concatenate(accs, axis=1)


def _tc_knn_kernel(xt_ref, x_ref, out_ref, *, n, row_lo, r):
    xt = xt_ref[0]
    x6 = x_ref[0]
    for h in range(2):
        xt3 = lax.slice(xt, (0, 3 * h), (n, 3 * h + 3))
        c3 = lax.slice(x6, (3 * h, 0), (3 * h + 3, n))
        out_ref[0, h] = _nbr_sum_tc(xt3, c3, n, row_lo, r)


def _sc_knn(xs, n):
    ns = _SC_ROWS
    nrw = ns * 16 // _NWORKERS
    nchunks = n // 16
    mesh = plsc.VectorSubcoreMesh(core_axis_name="c", subcore_axis_name="s")

    @functools.partial(
        pl.kernel, mesh=mesh,
        compiler_params=pltpu.CompilerParams(needs_layout_passes=False),
        out_type=jax.ShapeDtypeStruct((16 * 3 * ns,), jnp.float32),
        scratch_types=[
            pltpu.VMEM((n,), jnp.float32),
            pltpu.VMEM((n,), jnp.float32),
            pltpu.VMEM((n,), jnp.float32),
            pltpu.VMEM((n,), jnp.float32),
            pltpu.VMEM((n,), jnp.float32),
            pltpu.VMEM((n,), jnp.float32),
            pltpu.VMEM((n,), jnp.float32),
            pltpu.VMEM((nrw,), jnp.float32),
            pltpu.VMEM((nrw,), jnp.float32),
            pltpu.VMEM((nrw,), jnp.float32),
        ],
    )
    def k(xs_hbm, out_hbm, ce0, ce1, ce2, cb0, cb1, cb2, xxv, o0, o1, o2):
        wid = lax.axis_index("s") * 2 + lax.axis_index("c")
        bh = wid // 2
        sub = wid % 2
        row0 = sub * nrw
        pltpu.sync_copy(xs_hbm.at[pl.ds((bh * 3 + 0) * n, n)], ce0)
        pltpu.sync_copy(xs_hbm.at[pl.ds((bh * 3 + 1) * n, n)], ce1)
        pltpu.sync_copy(xs_hbm.at[pl.ds((bh * 3 + 2) * n, n)], ce2)

        def rnbf16(v):
            bits = plsc.bitcast(v, jnp.uint32)
            r = bits + jnp.uint32(0x7FFF) + ((bits >> 16) & jnp.uint32(1))
            return plsc.bitcast(r & jnp.uint32(0xFFFF0000), jnp.float32)

        def pre(i, _):
            ds = pl.ds(i * 16, 16)
            v0 = ce0[ds]
            v1 = ce1[ds]
            v2 = ce2[ds]
            cb0[ds] = rnbf16(v0)
            cb1[ds] = rnbf16(v1)
            cb2[ds] = rnbf16(v2)
            xxv[ds] = v0 * v0 + v1 * v1 + v2 * v2
            return 0

        lax.fori_loop(0, nchunks, pre, 0)

        lanes = lax.broadcasted_iota(jnp.int32, (16,), 0)
        kmask = lanes < _K
        zero = jnp.float32(0.0)

        neg = jnp.full((16,), -3e38, jnp.float32)
        zi = jnp.zeros((16,), jnp.int32)

        def group_body(g, _):
            def row_body(j, res):
                res0, res1, res2 = res
                base = row0 + g * 16 + 4 * j
                rows = []
                for i in range(4):
                    rr = jnp.full((16,), base + i, jnp.int32)
                    rows.append((plsc.load_gather(cb0, [rr]),
                                 plsc.load_gather(cb1, [rr]),
                                 plsc.load_gather(cb2, [rr]),
                                 -plsc.load_gather(xxv, [rr])))

                def merge16(vals, idxs, key, cols):
                    sk, si = plsc.sort_key_val(key, cols, descending=True)
                    rk = lax.rev(sk, (0,))
                    ri = lax.rev(si, (0,))
                    take = vals >= rk
                    nv = jnp.where(take, vals, rk)
                    ni = jnp.where(take, idxs, ri)
                    nv, ni = plsc.sort_key_val(nv, ni, descending=True)
                    return nv, ni

                def chunk_body(cc, carry):
                    ds = pl.ds(cc * 16, 16)
                    c0 = cb0[ds]
                    c1 = cb1[ds]
                    c2 = cb2[ds]
                    xc = xxv[ds]
                    cols = lanes + cc * 16
                    out = []
                    for i in range(4):
                        vx0, vx1, vx2, nxx = rows[i]
                        key = nxx - jnp.float32(-2.0) * (
                            vx0 * c0 + vx1 * c1 + vx2 * c2) - xc
                        v, ix = merge16(carry[2 * i], carry[2 * i + 1],
                                        key, cols)
                        out.extend((v, ix))
                    return tuple(out)

                tops = lax.fori_loop(
                    0, nchunks, chunk_body, (neg, zi) * 4)

                for (idxs, isj) in [(tops[2 * i + 1], lanes == 4 * j + i)
                                    for i in range(4)]:
                    g0 = plsc.load_gather(ce0, [idxs])
                    g1 = plsc.load_gather(ce1, [idxs])
                    g2 = plsc.load_gather(ce2, [idxs])
                    res0 = jnp.where(
                        isj, jnp.sum(jnp.where(kmask, g0, zero)), res0)
                    res1 = jnp.where(
                        isj, jnp.sum(jnp.where(kmask, g1, zero)), res1)
                    res2 = jnp.where(
                        isj, jnp.sum(jnp.where(kmask, g2, zero)), res2)
                return res0, res1, res2

            res0, res1, res2 = lax.fori_loop(
                0, 4, row_body,
                (jnp.zeros((16,), jnp.float32),
                 jnp.zeros((16,), jnp.float32),
                 jnp.zeros((16,), jnp.float32)))
            ds = pl.ds(g * 16, 16)
            o0[ds] = res0
            o1[ds] = res1
            o2[ds] = res2
            return 0

        lax.fori_loop(0, nrw // 16, group_body, 0)

        pltpu.sync_copy(o0, out_hbm.at[pl.ds((bh * 3 + 0) * ns + row0, nrw)])
        pltpu.sync_copy(o1, out_hbm.at[pl.ds((bh * 3 + 1) * ns + row0, nrw)])
        pltpu.sync_copy(o2, out_hbm.at[pl.ds((bh * 3 + 2) * ns + row0, nrw)])

    return k(xs)


def _edge_head(nm, c3, w_c, b_c):
    wa = lax.slice(w_c, (0, 0), (w_c.shape[0], 3))
    wb = lax.slice(w_c, (0, 3), (w_c.shape[0], 6))
    return _dot(wa, nm) + _dot(wb - wa, c3) + b_c


def _mlp3(h, w1, b1, g1, be1, w2, b2, g2, be2, w3, b3):
    h = _gn_gelu(_dotb(w1, h) + b1, g1, be1)
    h = _gn_gelu(_dotb(w2, h) + b2, g2, be2)
    return _dotb(w3, h) + b3


def _dense_kernel(x_ref, scs_ref, tcs_ref, wcd_ref, bcd_ref, wcm_ref, bcm_ref,
                  wd1_ref, bd1_ref, gd1_ref, bed1_ref,
                  wd2_ref, bd2_ref, gd2_ref, bed2_ref, wd3_ref, bd3_ref,
                  wm1_ref, bm1_ref, gm1_ref, bem1_ref,
                  wm2_ref, bm2_ref, gm2_ref, bem2_ref, wm3_ref, bm3_ref,
                  wg1_ref, bg1_ref, gg1_ref, beg1_ref, wg2_ref, bg2_ref,
                  out_ref, *, n):
    x6 = x_ref[0]
    inv_k = jnp.float32(1.0 / _K)
    nms = []
    for h in range(2):
        nm = jnp.concatenate([scs_ref[0, h], tcs_ref[0, h]], axis=1) * inv_k
        nms.append(nm)

    cd = lax.slice(x6, (0, 0), (3, n))
    cm = lax.slice(x6, (3, 0), (6, n))
    xd = _edge_head(nms[0], cd, wcd_ref[...], bcd_ref[...])
    xm = _edge_head(nms[1], cm, wcm_ref[...], bcm_ref[...])

    xdo = _mlp3(xd, wd1_ref[...], bd1_ref[...], gd1_ref[...], bed1_ref[...],
                wd2_ref[...], bd2_ref[...], gd2_ref[...], bed2_ref[...],
                wd3_ref[...], bd3_ref[...])
    xmo = _mlp3(xm, wm1_ref[...], bm1_ref[...], gm1_ref[...], bem1_ref[...],
                wm2_ref[...], bm2_ref[...], gm2_ref[...], bem2_ref[...],
                wm3_ref[...], bm3_ref[...])

    xc = jnp.concatenate([xdo, xmo], axis=0)
    h = _gn_gelu(_dotb(wg1_ref[...], xc) + bg1_ref[...],
                 gg1_ref[...], beg1_ref[...])
    out_ref[0] = _dotb(wg2_ref[...], h) + bg2_ref[...]


def kernel(x, w_cd, b_cd, w_cm, b_cm, w_d1, b_d1, g_d1, be_d1, w_d2, b_d2,
           g_d2, be_d2, w_d3, b_d3, w_m1, b_m1, g_m1, be_m1, w_m2, b_m2,
           g_m2, be_m2, w_m3, b_m3, w_g1, b_g1, g_g1, be_g1, w_g2, b_g2):
    B, C, N = x.shape
    ns = _SC_ROWS
    nt = N - ns
    R = nt // -(-nt // 1024)
    xt = jnp.transpose(x, (0, 2, 1))
    xs = x.reshape(B * 2 * 3 * N)

    scsum = _sc_knn(xs, N).reshape(B, 2, 3, ns)

    tcsum = pl.pallas_call(
        functools.partial(_tc_knn_kernel, n=N, row_lo=ns, r=R),
        grid=(B,),
        in_specs=[
            pl.BlockSpec((1, N, C), lambda b: (b, 0, 0)),
            pl.BlockSpec((1, C, N), lambda b: (b, 0, 0)),
        ],
        out_specs=pl.BlockSpec((1, 2, 3, nt), lambda b: (b, 0, 0, 0)),
        out_shape=jax.ShapeDtypeStruct((B, 2, 3, nt), jnp.float32),
        compiler_params=pltpu.CompilerParams(
            dimension_semantics=("arbitrary",)),
    )(xt, x)

    def col(v):
        return v.reshape(-1, 1)

    weights = (w_cd, col(b_cd), w_cm, col(b_cm),
               w_d1, col(b_d1), col(g_d1), col(be_d1),
               w_d2, col(b_d2), col(g_d2), col(be_d2), w_d3, col(b_d3),
               w_m1, col(b_m1), col(g_m1), col(be_m1),
               w_m2, col(b_m2), col(g_m2), col(be_m2), w_m3, col(b_m3),
               w_g1, col(b_g1), col(g_g1), col(be_g1), w_g2, col(b_g2))

    def wspec(v):
        nd = v.ndim
        return pl.BlockSpec(v.shape, lambda b, _nd=nd: (0,) * _nd)

    in_specs = [
        pl.BlockSpec((1, C, N), lambda b: (b, 0, 0)),
        pl.BlockSpec((1, 2, 3, ns), lambda b: (b, 0, 0, 0)),
        pl.BlockSpec((1, 2, 3, nt), lambda b: (b, 0, 0, 0)),
    ] + [wspec(v) for v in weights]

    out = pl.pallas_call(
        functools.partial(_dense_kernel, n=N),
        grid=(B,),
        in_specs=in_specs,
        out_specs=pl.BlockSpec((1, 512, N), lambda b: (b, 0, 0)),
        out_shape=jax.ShapeDtypeStruct((B, 512, N), jnp.float32),
        compiler_params=pltpu.CompilerParams(
            dimension_semantics=("arbitrary",)),
    )(x, scsum, tcsum, *weights)
    return out
